# baseline (speedup 1.0000x reference)
import jax
import jax.numpy as jnp
from jax import lax
from jax.experimental import pallas as pl
from jax.experimental.pallas import tpu as pltpu

_BT = 1024
_NCORES = 2
_NSTREAMS = 4


def _route_block(x_ref, w_router, lt_ref, wt_ref, et_ref):
    nexp = w_router.shape[0]
    logits_t = jax.lax.dot_general(
        w_router, x_ref[0],
        dimension_numbers=(((1,), (1,)), ((), ())),
        preferred_element_type=jnp.float32,
    )
    lt_ref[0] = logits_t
    idx = jax.lax.broadcasted_iota(jnp.int32, logits_t.shape, 0)
    m1 = jnp.max(logits_t, axis=0, keepdims=True)
    a1 = jnp.min(jnp.where(logits_t == m1, idx, nexp), axis=0, keepdims=True)
    masked = jnp.where(idx == a1, -jnp.inf, logits_t)
    m2 = jnp.max(masked, axis=0, keepdims=True)
    a2 = jnp.min(jnp.where(masked == m2, idx, nexp), axis=0, keepdims=True)
    w1 = 1.0 / (1.0 + jnp.exp(m2 - m1))
    w2 = 1.0 - w1
    wt_ref[0] = jnp.concatenate([w1, w2], axis=0)
    et_ref[0] = jnp.concatenate([a1, a2], axis=0)


def _core_body(x_hbm, w_hbm, lt_hbm, wt_hbm, et_hbm, w_vmem, wsem):
    b, s, h = x_hbm.shape
    jcnt = s // _BT // _NSTREAMS
    pltpu.make_async_copy(w_hbm, w_vmem, wsem).start()
    pltpu.make_async_copy(w_hbm, w_vmem, wsem).wait()
    w_router = w_vmem[...]

    def inner(*refs):
        x_refs = refs[:_NSTREAMS]
        lt_refs = refs[_NSTREAMS:2 * _NSTREAMS]
        wt_refs = refs[2 * _NSTREAMS:3 * _NSTREAMS]
        et_refs = refs[3 * _NSTREAMS:]
        for k in range(_NSTREAMS):
            _route_block(x_refs[k], w_router, lt_refs[k], wt_refs[k],
                         et_refs[k])

    in_specs = [
        pl.BlockSpec((1, _BT, h), (lambda i, j, k=k: (i, k * jcnt + j, 0)))
        for k in range(_NSTREAMS)
    ]
    out_specs = (
        [pl.BlockSpec((1, 8, _BT), (lambda i, j, k=k: (i, 0, k * jcnt + j)))
         for k in range(_NSTREAMS)]
        + [pl.BlockSpec((1, 2, _BT), (lambda i, j, k=k: (i, 0, k * jcnt + j)))
           for k in range(_NSTREAMS)]
        + [pl.BlockSpec((1, 2, _BT), (lambda i, j, k=k: (i, 0, k * jcnt + j)))
           for k in range(_NSTREAMS)]
    )
    pltpu.emit_pipeline(
        inner,
        grid=(b, jcnt),
        in_specs=in_specs,
        out_specs=out_specs,
        core_axis_name="core",
        dimension_semantics=(pltpu.PARALLEL, pltpu.ARBITRARY),
    )(*([x_hbm] * _NSTREAMS),
      *([lt_hbm] * _NSTREAMS),
      *([wt_hbm] * _NSTREAMS),
      *([et_hbm] * _NSTREAMS))


def kernel(hidden_states, W_router):
    b, s, h = hidden_states.shape
    n_exp = W_router.shape[0]

    mesh = pltpu.create_tensorcore_mesh("core", num_cores=_NCORES)
    run = pl.kernel(
        _core_body,
        out_type=[
            jax.ShapeDtypeStruct((b, n_exp, s), jnp.float32),
            jax.ShapeDtypeStruct((b, 2, s), jnp.float32),
            jax.ShapeDtypeStruct((b, 2, s), jnp.int32),
        ],
        mesh=mesh,
        scratch_types=[
            pltpu.VMEM((n_exp, h), jnp.float32),
            pltpu.SemaphoreType.DMA,
        ],
    )
    logits_t, weights_t, experts_t = run(hidden_states, W_router)

    return (
        weights_t.transpose(0, 2, 1),
        experts_t.transpose(0, 2, 1),
        logits_t.transpose(0, 2, 1),
    )


# speedup vs baseline: 1.0645x; 1.0645x over previous
import jax
import jax.numpy as jnp
from jax import lax
from jax.experimental import pallas as pl
from jax.experimental.pallas import tpu as pltpu

_BT = 512
_NCORES = 2
_NSTREAMS = 4


def _route_block(x_ref, w_router, lt_ref, wt_ref, et_ref):
    nexp = w_router.shape[0]
    logits_t = jax.lax.dot_general(
        w_router, x_ref[0],
        dimension_numbers=(((1,), (1,)), ((), ())),
        preferred_element_type=jnp.float32,
    )
    lt_ref[0] = logits_t
    idx = jax.lax.broadcasted_iota(jnp.int32, logits_t.shape, 0)
    m1 = jnp.max(logits_t, axis=0, keepdims=True)
    a1 = jnp.min(jnp.where(logits_t == m1, idx, nexp), axis=0, keepdims=True)
    masked = jnp.where(idx == a1, -jnp.inf, logits_t)
    m2 = jnp.max(masked, axis=0, keepdims=True)
    a2 = jnp.min(jnp.where(masked == m2, idx, nexp), axis=0, keepdims=True)
    w1 = 1.0 / (1.0 + jnp.exp(m2 - m1))
    w2 = 1.0 - w1
    wt_ref[0] = jnp.concatenate([w1, w2], axis=0)
    et_ref[0] = jnp.concatenate([a1, a2], axis=0)


def _core_body(x_hbm, w_hbm, lt_hbm, wt_hbm, et_hbm, w_vmem, wsem):
    b, s, h = x_hbm.shape
    jcnt = s // _BT // _NSTREAMS
    pltpu.make_async_copy(w_hbm, w_vmem, wsem).start()
    pltpu.make_async_copy(w_hbm, w_vmem, wsem).wait()
    w_router = w_vmem[...]

    def inner(*refs):
        x_refs = refs[:_NSTREAMS]
        lt_refs = refs[_NSTREAMS:2 * _NSTREAMS]
        wt_refs = refs[2 * _NSTREAMS:3 * _NSTREAMS]
        et_refs = refs[3 * _NSTREAMS:]
        for k in range(_NSTREAMS):
            _route_block(x_refs[k], w_router, lt_refs[k], wt_refs[k],
                         et_refs[k])

    in_specs = [
        pl.BlockSpec((1, _BT, h), (lambda i, j, k=k: (i, k * jcnt + j, 0)))
        for k in range(_NSTREAMS)
    ]
    out_specs = (
        [pl.BlockSpec((1, 8, _BT), (lambda i, j, k=k: (i, 0, k * jcnt + j)))
         for k in range(_NSTREAMS)]
        + [pl.BlockSpec((1, 2, _BT), (lambda i, j, k=k: (i, 0, k * jcnt + j)))
           for k in range(_NSTREAMS)]
        + [pl.BlockSpec((1, 2, _BT), (lambda i, j, k=k: (i, 0, k * jcnt + j)))
           for k in range(_NSTREAMS)]
    )
    pltpu.emit_pipeline(
        inner,
        grid=(b, jcnt),
        in_specs=in_specs,
        out_specs=out_specs,
        core_axis_name="core",
        dimension_semantics=(pltpu.PARALLEL, pltpu.ARBITRARY),
    )(*([x_hbm] * _NSTREAMS),
      *([lt_hbm] * _NSTREAMS),
      *([wt_hbm] * _NSTREAMS),
      *([et_hbm] * _NSTREAMS))


def kernel(hidden_states, W_router):
    b, s, h = hidden_states.shape
    n_exp = W_router.shape[0]

    mesh = pltpu.create_tensorcore_mesh("core", num_cores=_NCORES)
    run = pl.kernel(
        _core_body,
        out_type=[
            jax.ShapeDtypeStruct((b, n_exp, s), jnp.float32),
            jax.ShapeDtypeStruct((b, 2, s), jnp.float32),
            jax.ShapeDtypeStruct((b, 2, s), jnp.int32),
        ],
        mesh=mesh,
        scratch_types=[
            pltpu.VMEM((n_exp, h), jnp.float32),
            pltpu.SemaphoreType.DMA,
        ],
    )
    logits_t, weights_t, experts_t = run(hidden_states, W_router)

    return (
        weights_t.transpose(0, 2, 1),
        experts_t.transpose(0, 2, 1),
        logits_t.transpose(0, 2, 1),
    )
